# baseline (speedup 1.0000x reference)
import functools

import jax
import jax.numpy as jnp
from jax import lax
from jax.experimental import pallas as pl
from jax.experimental.pallas import tpu as pltpu
from jax.experimental.pallas import tpu_sc as plsc

_N = 10000
_E = 320000
_D = 128
_HD = 64
_PW = 144
_NL = 3
_EPS = 1e-12

_CORES = 2
_TILES = 16
_WORKERS = _CORES * _TILES
_EPW = _E // _WORKERS
_CH = 80
_NCHUNK = _EPW // _CH
_RPT = _N // _TILES
_ZR = 125


_sc_mesh = plsc.VectorSubcoreMesh(core_axis_name="c", subcore_axis_name="s")


@functools.partial(
    pl.kernel,
    mesh=_sc_mesh,
    compiler_params=pltpu.CompilerParams(
        use_tc_tiling_on_sc=False, needs_layout_passes=False),
    out_type=(
        jax.ShapeDtypeStruct((2, _CORES, _N, _HD), jnp.float32),
        jax.ShapeDtypeStruct((_CORES, _TILES, _N), jnp.float32),
    ),
    scratch_types=(
        [pltpu.VMEM((_CH,), jnp.int32)] * 2 +
        [pltpu.VMEM((_CH,), jnp.int32)] * 2 +
        [pltpu.VMEM((_CH,), jnp.int32)] * 2 +
        [pltpu.VMEM((_CH, _PW), jnp.float32)] * 2 +
        [pltpu.VMEM((_CH, _PW), jnp.float32)] * 2 +
        [pltpu.VMEM((_CH, _HD), jnp.float32)] * 2 +
        [
            pltpu.VMEM((_N,), jnp.float32),
            pltpu.VMEM((_EPW,), jnp.float32),
            pltpu.VMEM((_ZR, _HD), jnp.float32),
            pltpu.VMEM((16,), jnp.float32),
            pltpu.VMEM_SHARED((_N, _HD), jnp.float32),
        ] +
        [pltpu.SemaphoreType.DMA] * 8
    ),
)
def _edge_call(packed, src, dst, beta16, out, s_out,
               srcv0, srcv1, dstv0, dstv1, sdst0, sdst1,
               hs0, hs1, hd0, hd1, stg0, stg1,
               s_tile, w_tile, zbuf, beta_v, acc_sh,
               gs0, gs1, gd0, gd1, ss0, ss1, is0, is1):
    cid = lax.axis_index("c")
    sid = lax.axis_index("s")
    srcv = (srcv0, srcv1)
    dstv = (dstv0, dstv1)
    sdst = (sdst0, sdst1)
    is_b = (is0, is1)
    hs_b = (hs0, hs1)
    hd_b = (hd0, hd1)
    stg_b = (stg0, stg1)
    gs_b = (gs0, gs1)
    gd_b = (gd0, gd1)
    ss_b = (ss0, ss1)

    zero16 = jnp.zeros((16,), jnp.float32)

    def _zero_zbuf():
        def _zrow(r, _):
            for j in range(_HD // 16):
                zbuf[r, pl.ds(16 * j, 16)] = zero16
            return 0
        lax.fori_loop(0, _ZR, _zrow, 0)

    def _zero_acc_slice(r0):
        for kk in range(_RPT // _ZR):
            pltpu.sync_copy(zbuf, acc_sh.at[pl.ds(r0 + kk * _ZR, _ZR)])

    def _drain_acc_slice(r0, half):
        for kk in range(_RPT // _ZR):
            rr = r0 + kk * _ZR
            pltpu.sync_copy(acc_sh.at[pl.ds(rr, _ZR)], zbuf)
            pltpu.sync_copy(zbuf, out.at[half, cid, pl.ds(rr, _ZR)])

    _zero_zbuf()

    def _zs(r, _):
        s_tile[pl.ds(16 * r, 16)] = zero16
        return 0
    lax.fori_loop(0, _N // 16, _zs, 0)

    r0 = sid * _RPT
    _zero_acc_slice(r0)
    pltpu.sync_copy(beta16, beta_v)
    plsc.subcore_barrier()

    beta = beta_v[...]
    lane = lax.iota(jnp.int32, 16)
    lane0 = jnp.zeros((16,), jnp.int32)
    m0 = lane == 0

    def _hsum(v):
        for k in (1, 2, 4, 8):
            v = v + v.at[lane ^ k].get(mode="promise_in_bounds")
        return v

    wid = cid * _TILES + sid
    base = wid * _EPW

    def _issue_idx(k, b):
        eb = pl.multiple_of(base + k * _CH, _CH)
        pltpu.async_copy(src.at[pl.ds(eb, _CH)], srcv[b], is_b[b])
        pltpu.async_copy(dst.at[pl.ds(eb, _CH)], dstv[b], is_b[b])

    def _wait_idx(b):
        pltpu.make_async_copy(src.at[pl.ds(0, _CH)], srcv[b], is_b[b]).wait()
        pltpu.make_async_copy(dst.at[pl.ds(0, _CH)], dstv[b], is_b[b]).wait()

    def _issue_gathers(b, with_dst_rows):
        _wait_idx(b)
        pltpu.async_copy(packed.at[srcv[b]], hs_b[b], gs_b[b])
        if with_dst_rows:
            pltpu.async_copy(packed.at[dstv[b]], hd_b[b], gd_b[b])

    def _wait_gathers(b, with_dst_rows):
        pltpu.make_async_copy(packed.at[srcv[b]], hs_b[b], gs_b[b]).wait()
        if with_dst_rows:
            pltpu.make_async_copy(packed.at[dstv[b]], hd_b[b], gd_b[b]).wait()

    def _wait_scatter(b):
        pltpu.make_async_copy(stg_b[b], acc_sh.at[sdst[b]], ss_b[b]).wait()

    def _issue_scatter(k, b):
        for j in range(_CH // 16):
            sdst[b][pl.ds(16 * j, 16)] = dstv[b][pl.ds(16 * j, 16)]
        pltpu.async_copy(stg_b[b], acc_sh.at[sdst[b]], ss_b[b], add=True)

    def _compute1(k, b):
        hs, hd, stg, dsv = hs_b[b], hd_b[b], stg_b[b], dstv[b]
        wb = pl.multiple_of(k * _CH, _CH)

        for g in range(_CH // 16):
            def _edge(i, wgrp, g=g):
                e = g * 16 + i
                a = [hs[e, pl.ds(16 * j, 16)] for j in range(_HD // 16)]
                a8 = hs[e, pl.ds(_D, 16)]
                prods = [hs[e, pl.ds(16 * j, 16)] * hd[e, pl.ds(16 * j, 16)]
                         for j in range(_HD // 16, _D // 16)]
                prods += [a[j] * hd[e, pl.ds(16 * j, 16)]
                          for j in range(_HD // 16)]
                while len(prods) > 1:
                    prods = [prods[i] + prods[i + 1]
                             for i in range(0, len(prods), 2)]
                t = _hsum(prods[0])
                wv = jnp.exp(beta * t)
                sv = wv * a8.at[lane0].get(mode="promise_in_bounds")
                for j in range(_HD // 16):
                    stg[e, pl.ds(16 * j, 16)] = a[j] * sv
                return jnp.where(lane == i, wv, wgrp)

            wgrp = lax.fori_loop(0, 16, _edge, zero16)
            dsts = dsv[pl.ds(16 * g, 16)]
            plsc.addupdate_scatter(s_tile, [dsts], wgrp)
            w_tile[pl.ds(wb + 16 * g, 16)] = wgrp

    def _compute2(k, b):
        hs, stg = hs_b[b], stg_b[b]
        wb = pl.multiple_of(k * _CH, _CH)

        for g in range(_CH // 16):
            wsl = w_tile[pl.ds(wb + 16 * g, 16)]

            def _edge(i, _, g=g, wsl=wsl):
                e = g * 16 + i
                wv = wsl.at[lane0 + i].get(mode="promise_in_bounds")
                csv = hs[e, pl.ds(_D, 16)]
                sv = wv * csv.at[lane0].get(mode="promise_in_bounds")
                for j in range(_HD // 16):
                    aj = hs[e, pl.ds(_HD + 16 * j, 16)]
                    stg[e, pl.ds(16 * j, 16)] = aj * sv
                return 0

            lax.fori_loop(0, 16, _edge, 0)

    def _phase(compute, with_dst_rows):
        _issue_idx(0, 0)
        _issue_gathers(0, with_dst_rows)
        _issue_idx(1, 1)

        def _pair(kk, _):
            k0 = 2 * kk

            @pl.when(k0 + 1 < _NCHUNK)
            def _():
                _issue_gathers(1, with_dst_rows)
            _wait_gathers(0, with_dst_rows)

            @pl.when(k0 >= 2)
            def _():
                _wait_scatter(0)
            compute(k0, 0)
            _issue_scatter(k0, 0)

            @pl.when(k0 + 2 < _NCHUNK)
            def _():
                _issue_idx(k0 + 2, 0)

            @pl.when(k0 + 1 < _NCHUNK)
            def _():
                @pl.when(k0 + 2 < _NCHUNK)
                def _():
                    _issue_gathers(0, with_dst_rows)
                _wait_gathers(1, with_dst_rows)

                @pl.when(k0 >= 1)
                def _():
                    _wait_scatter(1)
                compute(k0 + 1, 1)
                _issue_scatter(k0 + 1, 1)

                @pl.when(k0 + 3 < _NCHUNK)
                def _():
                    _issue_idx(k0 + 3, 1)
            return 0

        lax.fori_loop(0, (_NCHUNK + 1) // 2, _pair, 0)
        _wait_scatter(0)
        _wait_scatter(1)

    _phase(_compute1, True)
    plsc.subcore_barrier()

    _drain_acc_slice(r0, 0)
    _zero_zbuf()
    _zero_acc_slice(r0)
    plsc.subcore_barrier()

    _phase(_compute2, False)
    plsc.subcore_barrier()

    _drain_acc_slice(r0, 1)
    pltpu.sync_copy(s_tile, s_out.at[cid, sid])


_BR = 1000


def _pack0_body(x_ref, o_ref):
    h = x_ref[...]
    ss = jnp.sum(h * h, axis=1, keepdims=True)
    c = jnp.sqrt(ss) + _EPS
    z = jnp.zeros((h.shape[0], _PW - _D - 1), jnp.float32)
    o_ref[...] = jnp.concatenate([h / c, c, z], axis=1)


_pack0 = pl.pallas_call(
    _pack0_body,
    grid=(_N // _BR,),
    in_specs=[pl.BlockSpec((_BR, _D), lambda i: (i, 0))],
    out_specs=pl.BlockSpec((_BR, _PW), lambda i: (i, 0)),
    out_shape=jax.ShapeDtypeStruct((_N, _PW), jnp.float32),
)


def _combine(q_lo_a, q_lo_b, q_hi_a, q_hi_b, sp):
    acc = jnp.concatenate([q_lo_a[0, 0] + q_lo_b[0, 0],
                           q_hi_a[0, 0] + q_hi_b[0, 0]], axis=1)
    s = jnp.sum(sp, axis=1, keepdims=True)
    return acc * (1.0 / (s + _EPS))


def _norm_body(qla, qlb, qha, qhb, sp_ref, o_ref):
    h = _combine(qla, qlb, qha, qhb, sp_ref[...])
    ss = jnp.sum(h * h, axis=1, keepdims=True)
    c = jnp.sqrt(ss) + _EPS
    z = jnp.zeros((h.shape[0], _PW - _D - 1), jnp.float32)
    o_ref[...] = jnp.concatenate([h / c, c, z], axis=1)


_norm = pl.pallas_call(
    _norm_body,
    grid=(_N // _BR,),
    in_specs=[
        pl.BlockSpec((1, 1, _BR, _HD), lambda i: (0, 0, i, 0)),
        pl.BlockSpec((1, 1, _BR, _HD), lambda i: (0, 1, i, 0)),
        pl.BlockSpec((1, 1, _BR, _HD), lambda i: (1, 0, i, 0)),
        pl.BlockSpec((1, 1, _BR, _HD), lambda i: (1, 1, i, 0)),
        pl.BlockSpec((_BR, _WORKERS), lambda i: (i, 0)),
    ],
    out_specs=pl.BlockSpec((_BR, _PW), lambda i: (i, 0)),
    out_shape=jax.ShapeDtypeStruct((_N, _PW), jnp.float32),
)


def _final_body(qla, qlb, qha, qhb, sp_ref, w_ref, b_ref, o_ref):
    h = _combine(qla, qlb, qha, qhb, sp_ref[...])
    hg = jnp.mean(h, axis=0, keepdims=True)
    o_ref[...] = lax.dot_general(
        hg, w_ref[...], (((1,), (0,)), ((), ())),
        preferred_element_type=jnp.float32,
        precision=lax.Precision.HIGHEST,
    ) + b_ref[...]


def _final(q, sp, w, b2):
    nc = w.shape[1]
    return pl.pallas_call(
        _final_body,
        grid=(1,),
        in_specs=[
            pl.BlockSpec((1, 1, _N, _HD), lambda i: (0, 0, 0, 0)),
            pl.BlockSpec((1, 1, _N, _HD), lambda i: (0, 1, 0, 0)),
            pl.BlockSpec((1, 1, _N, _HD), lambda i: (1, 0, 0, 0)),
            pl.BlockSpec((1, 1, _N, _HD), lambda i: (1, 1, 0, 0)),
            pl.BlockSpec((_N, _WORKERS), lambda i: (0, 0)),
            pl.BlockSpec((_D, nc), lambda i: (0, 0)),
            pl.BlockSpec((1, nc), lambda i: (0, 0)),
        ],
        out_specs=pl.BlockSpec((1, nc), lambda i: (0, 0)),
        out_shape=jax.ShapeDtypeStruct((1, nc), jnp.float32),
    )(q, q, q, q, sp, w, b2)


def kernel(x, edge_index, betas, W, b):
    src = edge_index[0]
    dst = edge_index[1]
    p = _pack0(x)
    q = sp = None
    for i in range(_NL):
        beta16 = jnp.full((16,), betas[i], jnp.float32)
        q, sp = _edge_call(p, src, dst, beta16)
        sp = sp.reshape(_WORKERS, _N).T
        if i < _NL - 1:
            p = _norm(q, q, q, q, sp)
    return _final(q, sp, W, b.reshape(1, -1))


# speedup vs baseline: 1.0398x; 1.0398x over previous
import functools

import jax
import jax.numpy as jnp
from jax import lax
from jax.experimental import pallas as pl
from jax.experimental.pallas import tpu as pltpu
from jax.experimental.pallas import tpu_sc as plsc

_N = 10000
_E = 320000
_D = 128
_HD = 64
_PW = 80
_NL = 3
_EPS = 1e-12

_CORES = 2
_TILES = 16
_WORKERS = _CORES * _TILES
_EPW = _E // _WORKERS
_CH = 80
_NCHUNK = _EPW // _CH
_RPT = _N // _TILES
_ZR = 125


_sc_mesh = plsc.VectorSubcoreMesh(core_axis_name="c", subcore_axis_name="s")


@functools.partial(
    pl.kernel,
    mesh=_sc_mesh,
    compiler_params=pltpu.CompilerParams(
        use_tc_tiling_on_sc=False, needs_layout_passes=False),
    out_type=(
        jax.ShapeDtypeStruct((2, _CORES, _N, _HD), jnp.float32),
        jax.ShapeDtypeStruct((_CORES, _TILES, _N), jnp.float32),
    ),
    scratch_types=(
        [pltpu.VMEM((_CH,), jnp.int32)] * 2 +
        [pltpu.VMEM((_CH,), jnp.int32)] * 2 +
        [pltpu.VMEM((_CH,), jnp.int32)] * 2 +
        [pltpu.VMEM((_CH, _PW), jnp.float32)] * 2 +
        [pltpu.VMEM((_CH, _PW), jnp.float32)] * 2 +
        [pltpu.VMEM((_CH, _HD), jnp.float32)] * 2 +
        [
            pltpu.VMEM((_N,), jnp.float32),
            pltpu.VMEM((_EPW,), jnp.float32),
            pltpu.VMEM((_ZR, _HD), jnp.float32),
            pltpu.VMEM((16,), jnp.float32),
            pltpu.VMEM_SHARED((_N, _HD), jnp.float32),
        ] +
        [pltpu.SemaphoreType.DMA] * 8
    ),
)
def _edge_call(packed, src, dst, beta16, out, s_out,
               srcv0, srcv1, dstv0, dstv1, sdst0, sdst1,
               hs0, hs1, hd0, hd1, stg0, stg1,
               s_tile, w_tile, zbuf, beta_v, acc_sh,
               gs0, gs1, gd0, gd1, ss0, ss1, is0, is1):
    cid = lax.axis_index("c")
    sid = lax.axis_index("s")
    srcv = (srcv0, srcv1)
    dstv = (dstv0, dstv1)
    sdst = (sdst0, sdst1)
    is_b = (is0, is1)
    hs_b = (hs0, hs1)
    hd_b = (hd0, hd1)
    stg_b = (stg0, stg1)
    gs_b = (gs0, gs1)
    gd_b = (gd0, gd1)
    ss_b = (ss0, ss1)

    zero16 = jnp.zeros((16,), jnp.float32)

    def _zero_zbuf():
        def _zrow(r, _):
            for j in range(_HD // 16):
                zbuf[r, pl.ds(16 * j, 16)] = zero16
            return 0
        lax.fori_loop(0, _ZR, _zrow, 0)

    def _zero_acc_slice(r0):
        for kk in range(_RPT // _ZR):
            pltpu.sync_copy(zbuf, acc_sh.at[pl.ds(r0 + kk * _ZR, _ZR)])

    def _drain_acc_slice(r0, half):
        for kk in range(_RPT // _ZR):
            rr = r0 + kk * _ZR
            pltpu.sync_copy(acc_sh.at[pl.ds(rr, _ZR)], zbuf)
            pltpu.sync_copy(zbuf, out.at[half, cid, pl.ds(rr, _ZR)])

    _zero_zbuf()

    def _zs(r, _):
        s_tile[pl.ds(16 * r, 16)] = zero16
        return 0
    lax.fori_loop(0, _N // 16, _zs, 0)

    r0 = sid * _RPT
    _zero_acc_slice(r0)
    pltpu.sync_copy(beta16, beta_v)
    plsc.subcore_barrier()

    beta = beta_v[...]
    lane = lax.iota(jnp.int32, 16)
    lane0 = jnp.zeros((16,), jnp.int32)
    m0 = lane == 0

    def _hsum(v):
        for k in (1, 2, 4, 8):
            v = v + v.at[lane ^ k].get(mode="promise_in_bounds")
        return v

    wid = cid * _TILES + sid
    base = wid * _EPW

    def _issue_idx(k, b):
        eb = pl.multiple_of(base + k * _CH, _CH)
        pltpu.async_copy(src.at[pl.ds(eb, _CH)], srcv[b], is_b[b])
        pltpu.async_copy(dst.at[pl.ds(eb, _CH)], dstv[b], is_b[b])

    def _wait_idx(b):
        pltpu.make_async_copy(src.at[pl.ds(0, _CH)], srcv[b], is_b[b]).wait()
        pltpu.make_async_copy(dst.at[pl.ds(0, _CH)], dstv[b], is_b[b]).wait()

    def _issue_gathers(b, with_dst_rows):
        _wait_idx(b)
        pltpu.async_copy(packed.at[srcv[b]], hs_b[b], gs_b[b])
        if with_dst_rows:
            pltpu.async_copy(packed.at[dstv[b]], hd_b[b], gd_b[b])

    def _wait_gathers(b, with_dst_rows):
        pltpu.make_async_copy(packed.at[srcv[b]], hs_b[b], gs_b[b]).wait()
        if with_dst_rows:
            pltpu.make_async_copy(packed.at[dstv[b]], hd_b[b], gd_b[b]).wait()

    def _wait_scatter(b):
        pltpu.make_async_copy(stg_b[b], acc_sh.at[sdst[b]], ss_b[b]).wait()

    def _issue_scatter(k, b):
        for j in range(_CH // 16):
            sdst[b][pl.ds(16 * j, 16)] = dstv[b][pl.ds(16 * j, 16)]
        pltpu.async_copy(stg_b[b], acc_sh.at[sdst[b]], ss_b[b], add=True)

    def _compute1(k, b):
        hs, hd, stg, dsv = hs_b[b], hd_b[b], stg_b[b], dstv[b]
        wb = pl.multiple_of(k * _CH, _CH)

        for g in range(_CH // 16):
            def _edge(i, wgrp, g=g):
                e = g * 16 + i
                dv = None
                slos = []
                for q in range(_HD // 16):
                    ws = hs[e, pl.ds(16 * q, 16)]
                    wd = hd[e, pl.ds(16 * q, 16)]
                    slo, shi = plsc.unpack(
                        plsc.bitcast(ws, jnp.bfloat16),
                        format=plsc.PackFormat.INTERLEAVED)
                    dlo, dhi = plsc.unpack(
                        plsc.bitcast(wd, jnp.bfloat16),
                        format=plsc.PackFormat.INTERLEAVED)
                    slos.append(slo)
                    p = slo * dlo + shi * dhi
                    dv = p if dv is None else dv + p
                t = _hsum(dv)
                wv = jnp.exp(beta * t)
                csv = hs[e, pl.ds(_HD, 16)]
                sv = wv * csv.at[lane0].get(mode="promise_in_bounds")
                for q in range(_HD // 16):
                    stg[e, pl.ds(16 * q, 16)] = slos[q] * sv
                return jnp.where(lane == i, wv, wgrp)

            wgrp = lax.fori_loop(0, 16, _edge, zero16)
            dsts = dsv[pl.ds(16 * g, 16)]
            plsc.addupdate_scatter(s_tile, [dsts], wgrp)
            w_tile[pl.ds(wb + 16 * g, 16)] = wgrp

    def _compute2(k, b):
        hs, stg = hs_b[b], stg_b[b]
        wb = pl.multiple_of(k * _CH, _CH)

        for g in range(_CH // 16):
            wsl = w_tile[pl.ds(wb + 16 * g, 16)]

            def _edge(i, _, g=g, wsl=wsl):
                e = g * 16 + i
                wv = wsl.at[lane0 + i].get(mode="promise_in_bounds")
                csv = hs[e, pl.ds(_HD, 16)]
                sv = wv * csv.at[lane0].get(mode="promise_in_bounds")
                for q in range(_HD // 16):
                    ws = hs[e, pl.ds(16 * q, 16)]
                    _, shi = plsc.unpack(
                        plsc.bitcast(ws, jnp.bfloat16),
                        format=plsc.PackFormat.INTERLEAVED)
                    stg[e, pl.ds(16 * q, 16)] = shi * sv
                return 0

            lax.fori_loop(0, 16, _edge, 0)

    def _phase(compute, with_dst_rows):
        _issue_idx(0, 0)
        _issue_gathers(0, with_dst_rows)
        _issue_idx(1, 1)

        def _pair(kk, _):
            k0 = 2 * kk

            @pl.when(k0 + 1 < _NCHUNK)
            def _():
                _issue_gathers(1, with_dst_rows)
            _wait_gathers(0, with_dst_rows)

            @pl.when(k0 >= 2)
            def _():
                _wait_scatter(0)
            compute(k0, 0)
            _issue_scatter(k0, 0)

            @pl.when(k0 + 2 < _NCHUNK)
            def _():
                _issue_idx(k0 + 2, 0)

            @pl.when(k0 + 1 < _NCHUNK)
            def _():
                @pl.when(k0 + 2 < _NCHUNK)
                def _():
                    _issue_gathers(0, with_dst_rows)
                _wait_gathers(1, with_dst_rows)

                @pl.when(k0 >= 1)
                def _():
                    _wait_scatter(1)
                compute(k0 + 1, 1)
                _issue_scatter(k0 + 1, 1)

                @pl.when(k0 + 3 < _NCHUNK)
                def _():
                    _issue_idx(k0 + 3, 1)
            return 0

        lax.fori_loop(0, (_NCHUNK + 1) // 2, _pair, 0)
        _wait_scatter(0)
        _wait_scatter(1)

    _phase(_compute1, True)
    plsc.subcore_barrier()

    _drain_acc_slice(r0, 0)
    _zero_zbuf()
    _zero_acc_slice(r0)
    plsc.subcore_barrier()

    _phase(_compute2, False)
    plsc.subcore_barrier()

    _drain_acc_slice(r0, 1)
    pltpu.sync_copy(s_tile, s_out.at[cid, sid])


_BR = 1000


def _pack_rows(hn, c):
    lo = lax.bitcast_convert_type(
        hn[:, :_HD].astype(jnp.bfloat16), jnp.uint16).astype(jnp.uint32)
    hi = lax.bitcast_convert_type(
        hn[:, _HD:].astype(jnp.bfloat16), jnp.uint16).astype(jnp.uint32)
    w = lax.bitcast_convert_type(lo | (hi << 16), jnp.float32)
    z = jnp.zeros((hn.shape[0], _PW - _HD - 1), jnp.float32)
    return jnp.concatenate([w, c, z], axis=1)


def _pack0_body(x_ref, o_ref):
    h = x_ref[...]
    ss = jnp.sum(h * h, axis=1, keepdims=True)
    c = jnp.sqrt(ss) + _EPS
    o_ref[...] = _pack_rows(h / c, c)


_pack0 = pl.pallas_call(
    _pack0_body,
    grid=(_N // _BR,),
    in_specs=[pl.BlockSpec((_BR, _D), lambda i: (i, 0))],
    out_specs=pl.BlockSpec((_BR, _PW), lambda i: (i, 0)),
    out_shape=jax.ShapeDtypeStruct((_N, _PW), jnp.float32),
)


def _combine(q_lo_a, q_lo_b, q_hi_a, q_hi_b, sp):
    acc = jnp.concatenate([q_lo_a[0, 0] + q_lo_b[0, 0],
                           q_hi_a[0, 0] + q_hi_b[0, 0]], axis=1)
    s = jnp.sum(sp, axis=1, keepdims=True)
    return acc * (1.0 / (s + _EPS))


def _norm_body(qla, qlb, qha, qhb, sp_ref, o_ref):
    h = _combine(qla, qlb, qha, qhb, sp_ref[...])
    ss = jnp.sum(h * h, axis=1, keepdims=True)
    c = jnp.sqrt(ss) + _EPS
    o_ref[...] = _pack_rows(h / c, c)


_norm = pl.pallas_call(
    _norm_body,
    grid=(_N // _BR,),
    in_specs=[
        pl.BlockSpec((1, 1, _BR, _HD), lambda i: (0, 0, i, 0)),
        pl.BlockSpec((1, 1, _BR, _HD), lambda i: (0, 1, i, 0)),
        pl.BlockSpec((1, 1, _BR, _HD), lambda i: (1, 0, i, 0)),
        pl.BlockSpec((1, 1, _BR, _HD), lambda i: (1, 1, i, 0)),
        pl.BlockSpec((_BR, _WORKERS), lambda i: (i, 0)),
    ],
    out_specs=pl.BlockSpec((_BR, _PW), lambda i: (i, 0)),
    out_shape=jax.ShapeDtypeStruct((_N, _PW), jnp.float32),
)


def _final_body(qla, qlb, qha, qhb, sp_ref, w_ref, b_ref, o_ref):
    h = _combine(qla, qlb, qha, qhb, sp_ref[...])
    hg = jnp.mean(h, axis=0, keepdims=True)
    o_ref[...] = lax.dot_general(
        hg, w_ref[...], (((1,), (0,)), ((), ())),
        preferred_element_type=jnp.float32,
        precision=lax.Precision.HIGHEST,
    ) + b_ref[...]


def _final(q, sp, w, b2):
    nc = w.shape[1]
    return pl.pallas_call(
        _final_body,
        grid=(1,),
        in_specs=[
            pl.BlockSpec((1, 1, _N, _HD), lambda i: (0, 0, 0, 0)),
            pl.BlockSpec((1, 1, _N, _HD), lambda i: (0, 1, 0, 0)),
            pl.BlockSpec((1, 1, _N, _HD), lambda i: (1, 0, 0, 0)),
            pl.BlockSpec((1, 1, _N, _HD), lambda i: (1, 1, 0, 0)),
            pl.BlockSpec((_N, _WORKERS), lambda i: (0, 0)),
            pl.BlockSpec((_D, nc), lambda i: (0, 0)),
            pl.BlockSpec((1, nc), lambda i: (0, 0)),
        ],
        out_specs=pl.BlockSpec((1, nc), lambda i: (0, 0)),
        out_shape=jax.ShapeDtypeStruct((1, nc), jnp.float32),
    )(q, q, q, q, sp, w, b2)


def kernel(x, edge_index, betas, W, b):
    src = edge_index[0]
    dst = edge_index[1]
    p = _pack0(x)
    q = sp = None
    for i in range(_NL):
        beta16 = jnp.full((16,), betas[i], jnp.float32)
        q, sp = _edge_call(p, src, dst, beta16)
        sp = sp.reshape(_WORKERS, _N).T
        if i < _NL - 1:
            p = _norm(q, q, q, q, sp)
    return _final(q, sp, W, b.reshape(1, -1))
